# Initial kernel scaffold; baseline (speedup 1.0000x reference)
import jax
import jax.numpy as jnp
from jax.experimental import pallas as pl


def kernel(x, edge_index, W1, b1, gamma, beta, run_mean, run_var, W2, b2):
    raise NotImplementedError("write your pallas kernel here")


# speedup vs baseline: 1.0133x; 1.0133x over previous
import functools

import jax
import jax.numpy as jnp
import numpy as np
from jax.experimental import pallas as pl
from jax.experimental.pallas import tpu as pltpu

N = 10000
E = 320000
D_IN = 128
HID = 128
OUT = 64
K = 10
ALPHA = 0.1
NUM_LAYERS = 2
EPS = 1e-5

ROWS_BLK = 400
GRID_ROWS = N // ROWS_BLK


def _linear_kernel(x_ref, wt_ref, b_ref, o_ref):
    o_ref[...] = (
        jnp.dot(x_ref[...], wt_ref[...], preferred_element_type=jnp.float32)
        + b_ref[...]
    )


def _linear(x, wt, b):
    dout = wt.shape[1]
    return pl.pallas_call(
        _linear_kernel,
        grid=(GRID_ROWS,),
        in_specs=[
            pl.BlockSpec((ROWS_BLK, x.shape[1]), lambda i: (i, 0)),
            pl.BlockSpec((x.shape[1], dout), lambda i: (0, 0)),
            pl.BlockSpec((1, dout), lambda i: (0, 0)),
        ],
        out_specs=pl.BlockSpec((ROWS_BLK, dout), lambda i: (i, 0)),
        out_shape=jax.ShapeDtypeStruct((N, dout), jnp.float32),
    )(x, wt, b)


def _head_kernel(h_ref, wt_ref, b_ref, emb_ref, out_ref):
    emb = (
        jnp.dot(h_ref[...], wt_ref[...], preferred_element_type=jnp.float32)
        + b_ref[...]
    )
    emb_ref[...] = emb
    m = jnp.max(emb, axis=1, keepdims=True)
    lse = m + jnp.log(jnp.sum(jnp.exp(emb - m), axis=1, keepdims=True))
    out_ref[...] = emb - lse


def _head(h, wt, b):
    return pl.pallas_call(
        _head_kernel,
        grid=(GRID_ROWS,),
        in_specs=[
            pl.BlockSpec((ROWS_BLK, HID), lambda i: (i, 0)),
            pl.BlockSpec((HID, OUT), lambda i: (0, 0)),
            pl.BlockSpec((1, OUT), lambda i: (0, 0)),
        ],
        out_specs=[
            pl.BlockSpec((ROWS_BLK, OUT), lambda i: (i, 0)),
            pl.BlockSpec((ROWS_BLK, OUT), lambda i: (i, 0)),
        ],
        out_shape=[
            jax.ShapeDtypeStruct((N, OUT), jnp.float32),
            jax.ShapeDtypeStruct((N, OUT), jnp.float32),
        ],
    )(h, wt, b)


def kernel(x, edge_index, W1, b1, gamma, beta, run_mean, run_var, W2, b2):
    loop = jnp.arange(N, dtype=edge_index.dtype)
    src = jnp.concatenate([edge_index[0], loop])
    dst = jnp.concatenate([edge_index[1], loop])
    deg = jnp.zeros((N,), jnp.float32).at[dst].add(1.0)
    dinv = jax.lax.rsqrt(deg)
    norm = dinv[src] * dinv[dst]

    h = _linear(x, W1.T, b1[None, :])
    for i in range(NUM_LAYERS):
        h0 = h
        for _ in range(K):
            msg = norm[:, None] * h[src]
            agg = jnp.zeros_like(h).at[dst].add(msg)
            h = (1.0 - ALPHA) * agg + ALPHA * h0
        scale = gamma[i] * jax.lax.rsqrt(run_var[i] + EPS)
        shift = beta[i] - run_mean[i] * scale
        h = h * scale[None, :] + shift[None, :]
    emb, out = _head(h, W2.T, b2[None, :])
    return (out, emb)
